# baseline (speedup 1.0000x reference)
import jax
import jax.numpy as jnp
from jax import lax
from jax.experimental import pallas as pl
from jax.experimental.pallas import tpu as pltpu
from jax.experimental.pallas import tpu_sc as plsc

N = 10000
NP = 10240
E = 320000
SEQ = 8
HOR = 4
NC, NS = 2, 16
CHUNK = 128
CH = 160
NCHUNKS = NS * CH
E_PAD = NCHUNKS * CHUNK
CHE = NCHUNKS // (NC * NS)
RPT = NP // NS
ZR = NP // NS
TRASH = NP - N
BN = 2048
GRID = NP // BN

F32 = jnp.float32


def _make_sc_edge():
    mesh = plsc.VectorSubcoreMesh(core_axis_name="c", subcore_axis_name="s")
    CK = 64
    CPW = E_PAD // (NC * NS * CK)
    GRP = CPW // 8

    def body(table, srcg, dstg, zz, out, acc_sp, idx_v, rows_v, semg, semi):
        c = lax.axis_index("c")
        s = lax.axis_index("s")
        pltpu.sync_copy(zz, acc_sp.at[pl.ds(s * ZR, ZR), :])
        plsc.subcore_barrier()
        base = (c * NS + s) * CPW

        def gath(ring, row, buf):
            pltpu.async_copy(table.at[idx_v.at[ring, 0, row]], rows_v.at[buf], semg)

        def gath_wait(buf):
            pltpu.make_async_copy(table.at[idx_v.at[0, 0, 0]], rows_v.at[buf],
                                  semg).wait()

        pltpu.sync_copy(srcg.at[pl.ds(base, 8), :], idx_v.at[0, 0])
        pltpu.sync_copy(dstg.at[pl.ds(base, 8), :], idx_v.at[0, 1])
        gath(0, 0, 0)
        gath(0, 1, 1)
        gath(0, 2, 2)
        gath(0, 3, 3)

        @pl.loop(0, GRP)
        def _(g):
            r = lax.rem(g, 2)

            @pl.when(g < GRP - 1)
            def _():
                nb = base + (g + 1) * 8
                pltpu.async_copy(srcg.at[pl.ds(nb, 8), :], idx_v.at[1 - r, 0], semi)
                pltpu.async_copy(dstg.at[pl.ds(nb, 8), :], idx_v.at[1 - r, 1], semi)

            for j in range(8):
                b = j % 4
                gath_wait(b)
                pltpu.sync_copy(rows_v.at[b], acc_sp.at[idx_v.at[r, 1, j]], add=True)
                if j < 4:
                    gath(r, j + 4, b)
                else:
                    @pl.when(g < GRP - 1)
                    def _():
                        if j == 4:
                            pltpu.make_async_copy(srcg.at[pl.ds(base, 8), :],
                                                  idx_v.at[1 - r, 0], semi).wait()
                            pltpu.make_async_copy(dstg.at[pl.ds(base, 8), :],
                                                  idx_v.at[1 - r, 1], semi).wait()
                        gath(1 - r, j - 4, b)

        plsc.subcore_barrier()
        pltpu.sync_copy(acc_sp.at[pl.ds(s * RPT, RPT), :],
                        out.at[c, pl.ds(s * RPT, RPT), :])

    return pl.kernel(
        body,
        out_type=jax.ShapeDtypeStruct((NC, NP, 128), F32),
        mesh=mesh,
        scratch_types=[
            pltpu.VMEM_SHARED((NP, 128), F32),
            pltpu.VMEM((2, 2, 8, 64), jnp.int32),
            pltpu.VMEM((4, 64, 128), F32),
            pltpu.SemaphoreType.DMA,
            pltpu.SemaphoreType.DMA,
        ],
    )


def _make_sc_staged16():
    mesh = plsc.VectorSubcoreMesh(core_axis_name="c", subcore_axis_name="s")
    GRP = CHE // 8

    def body(table, srcg, dstg, zz, out, tab_sp, acc_sp, idx_v, rows_v, semg, semi):
        c = lax.axis_index("c")
        s = lax.axis_index("s")
        pltpu.sync_copy(table.at[pl.ds(s * RPT, RPT), :],
                        tab_sp.at[pl.ds(s * RPT, RPT), :])
        pltpu.sync_copy(zz, acc_sp.at[pl.ds(s * ZR, ZR), :])
        plsc.subcore_barrier()
        base = (c * NS + s) * CHE
        pltpu.sync_copy(srcg.at[pl.ds(base, 8), :], idx_v.at[0, 0])
        pltpu.sync_copy(dstg.at[pl.ds(base, 8), :], idx_v.at[0, 1])

        @pl.loop(0, GRP)
        def _(g):
            r = lax.rem(g, 2)

            @pl.when(g > 0)
            def _():
                pltpu.make_async_copy(srcg.at[pl.ds(base, 8), :],
                                      idx_v.at[r, 0], semi).wait()
                pltpu.make_async_copy(dstg.at[pl.ds(base, 8), :],
                                      idx_v.at[r, 1], semi).wait()

            pltpu.async_copy(tab_sp.at[idx_v.at[r, 0, 0]], rows_v.at[0], semg)

            @pl.when(g < GRP - 1)
            def _():
                nb = base + (g + 1) * 8
                pltpu.async_copy(srcg.at[pl.ds(nb, 8), :], idx_v.at[1 - r, 0], semi)
                pltpu.async_copy(dstg.at[pl.ds(nb, 8), :], idx_v.at[1 - r, 1], semi)

            for j in range(8):
                b = j % 2
                pltpu.make_async_copy(tab_sp.at[idx_v.at[r, 0, j]],
                                      rows_v.at[b], semg).wait()
                if j < 7:
                    pltpu.async_copy(tab_sp.at[idx_v.at[r, 0, j + 1]],
                                     rows_v.at[1 - b], semg)
                pltpu.sync_copy(rows_v.at[b], acc_sp.at[idx_v.at[r, 1, j]], add=True)

        plsc.subcore_barrier()
        pltpu.sync_copy(acc_sp.at[pl.ds(s * RPT, RPT), :],
                        out.at[c, pl.ds(s * RPT, RPT), :])

    return pl.kernel(
        body,
        out_type=jax.ShapeDtypeStruct((NC, NP, 16), F32),
        mesh=mesh,
        scratch_types=[
            pltpu.VMEM_SHARED((NP, 16), F32),
            pltpu.VMEM_SHARED((NP, 16), F32),
            pltpu.VMEM((2, 2, 8, CHUNK), jnp.int32),
            pltpu.VMEM((2, CHUNK, 16), F32),
            pltpu.SemaphoreType.DMA,
            pltpu.SemaphoreType.DMA,
        ],
    )


_SCE = _make_sc_edge()
_SC16 = _make_sc_staged16()


def _b3(k):
    return pl.BlockSpec((2, BN, k), lambda i: (0, i, 0))


def _b2(k):
    return pl.BlockSpec((BN, k), lambda i: (i, 0))


def _w(shape):
    nd = len(shape)
    return pl.BlockSpec(shape, lambda i, _n=nd: (0,) * _n)


def _dot(a, b):
    return jnp.dot(a, b, preferred_element_type=F32)


def _sl(v, b):
    return v[:, 64 * b:64 * b + 64]


def _gates_x_body(x, ax, h, m, wzx, wzh, uzx, uzh, bzr, rh, z):
    mm = m[0] + m[1]
    hv = h[...]
    for b in range(2):
        xb = x[:, 2 * b:2 * b + 2]
        axb = ax[:, 2 * b:2 * b + 2]
        zr = jax.nn.sigmoid(_dot(xb, wzx[b]) + _dot(_sl(hv, b), wzh[b])
                            + _dot(axb, uzx[b]) + _dot(_sl(mm, b), uzh[b]) + bzr[b])
        z[:, 64 * b:64 * b + 64] = zr[:, :64]
        rh[:, 64 * b:64 * b + 64] = zr[:, 64:] * _sl(hv, b)


def _gates_m_body(mt, h, ma, mb, wzm, wzh, uzm, uzh, bzr, rh, z):
    maa = ma[0] + ma[1]
    mbb = mb[0] + mb[1]
    mtv = mt[...]
    hv = h[...]
    for b in range(2):
        zr = jax.nn.sigmoid(_dot(_sl(mtv, b), wzm[b]) + _dot(_sl(hv, b), wzh[b])
                            + _dot(_sl(maa, b), uzm[b]) + _dot(_sl(mbb, b), uzh[b])
                            + bzr[b])
        z[:, 64 * b:64 * b + 64] = zr[:, :64]
        rh[:, 64 * b:64 * b + 64] = zr[:, 64:] * _sl(hv, b)


def _cand_x_body(x, ax, h, rh, m2, z, whx, whh, uhx, uhh, bh, wint, ap, hn_out, mt_out):
    m22 = m2[0] + m2[1]
    hv = h[...]
    rhv = rh[...]
    zv = z[...]
    hns = []
    for b in range(2):
        xb = x[:, 2 * b:2 * b + 2]
        axb = ax[:, 2 * b:2 * b + 2]
        hh = jnp.tanh(_dot(xb, whx[b]) + _dot(_sl(rhv, b), whh[b])
                      + _dot(axb, uhx[b]) + _dot(_sl(m22, b), uhh[b]) + bh[b])
        zb = _sl(zv, b)
        hns.append(zb * _sl(hv, b) + (1.0 - zb) * hh)
    for b in range(2):
        ib = jnp.tanh(_dot(hns[1 - b], wint[b]))
        v = hns[b] + ib
        hn_out[:, 64 * b:64 * b + 64] = v
        mt_out[:, 64 * b:64 * b + 64] = jnp.where(v >= 0, v, ap[...] * v)


def _cand_m_enc_body(mt, h, rh, ma, m4, z, whm, whh, uhm, uhh, bh, wint, hn_out):
    maa = ma[0] + ma[1]
    m44 = m4[0] + m4[1]
    mtv = mt[...]
    hv = h[...]
    rhv = rh[...]
    zv = z[...]
    hns = []
    for b in range(2):
        hh = jnp.tanh(_dot(_sl(mtv, b), whm[b]) + _dot(_sl(rhv, b), whh[b])
                      + _dot(_sl(maa, b), uhm[b]) + _dot(_sl(m44, b), uhh[b]) + bh[b])
        zb = _sl(zv, b)
        hns.append(zb * _sl(hv, b) + (1.0 - zb) * hh)
    for b in range(2):
        hn_out[:, 64 * b:64 * b + 64] = hns[b] + jnp.tanh(_dot(hns[1 - b], wint[b]))


def _cand_m_dec_body(mt, h, rh, ma, m4, z, whm, whh, uhm, uhh, bh, wint,
                     wout, bout, hn_out, out, s1t):
    maa = ma[0] + ma[1]
    m44 = m4[0] + m4[1]
    mtv = mt[...]
    hv = h[...]
    rhv = rh[...]
    zv = z[...]
    hns = []
    for b in range(2):
        hh = jnp.tanh(_dot(_sl(mtv, b), whm[b]) + _dot(_sl(rhv, b), whh[b])
                      + _dot(_sl(maa, b), uhm[b]) + _dot(_sl(m44, b), uhh[b]) + bh[b])
        zb = _sl(zv, b)
        hns.append(zb * _sl(hv, b) + (1.0 - zb) * hh)
    for b in range(2):
        v = hns[b] + jnp.tanh(_dot(hns[1 - b], wint[b]))
        hn_out[:, 64 * b:64 * b + 64] = v
        o = _dot(v, wout[b]) + bout[b]
        out[:, 2 * b:2 * b + 2] = o
        s1t[:, 8 * b:8 * b + 2] = o
        s1t[:, 8 * b + 2:8 * b + 8] = jnp.zeros((s1t.shape[0], 6), F32)


def _tc_gates_x(x, ax, h, m, w):
    return pl.pallas_call(
        _gates_x_body,
        grid=(GRID,),
        in_specs=[_b2(4), _b2(4), _b2(128), _b3(128),
                  _w((2, 2, 128)), _w((2, 64, 128)), _w((2, 2, 128)),
                  _w((2, 64, 128)), _w((2, 1, 128))],
        out_specs=[_b2(128), _b2(128)],
        out_shape=[jax.ShapeDtypeStruct((NP, 128), F32),
                   jax.ShapeDtypeStruct((NP, 128), F32)],
    )(x, ax, h, m, w["wzr_x"], w["wzr_h"], w["uzr_x"], w["uzr_h"], w["bzr"])


def _tc_gates_m(mt, h, ma, mb, w):
    return pl.pallas_call(
        _gates_m_body,
        grid=(GRID,),
        in_specs=[_b2(128), _b2(128), _b3(128), _b3(128),
                  _w((2, 64, 128)), _w((2, 64, 128)), _w((2, 64, 128)),
                  _w((2, 64, 128)), _w((2, 1, 128))],
        out_specs=[_b2(128), _b2(128)],
        out_shape=[jax.ShapeDtypeStruct((NP, 128), F32),
                   jax.ShapeDtypeStruct((NP, 128), F32)],
    )(mt, h, ma, mb, w["wzr_x"], w["wzr_h"], w["uzr_x"], w["uzr_h"], w["bzr"])


def _tc_cand_x(x, ax, h, rh, m2, z, w, wint, ap):
    return pl.pallas_call(
        _cand_x_body,
        grid=(GRID,),
        in_specs=[_b2(4), _b2(4), _b2(128), _b2(128), _b3(128), _b2(128),
                  _w((2, 2, 64)), _w((2, 64, 64)), _w((2, 2, 64)),
                  _w((2, 64, 64)), _w((2, 1, 64)), _w((2, 64, 64)), _w((1, 64))],
        out_specs=[_b2(128), _b2(128)],
        out_shape=[jax.ShapeDtypeStruct((NP, 128), F32),
                   jax.ShapeDtypeStruct((NP, 128), F32)],
    )(x, ax, h, rh, m2, z, w["wh_x"], w["wh_h"], w["uh_x"], w["uh_h"],
      w["bh"], wint, ap)


def _tc_cand_m_enc(mt, h, rh, ma, m4, z, w, wint):
    return pl.pallas_call(
        _cand_m_enc_body,
        grid=(GRID,),
        in_specs=[_b2(128), _b2(128), _b2(128), _b3(128), _b3(128), _b2(128),
                  _w((2, 64, 64)), _w((2, 64, 64)), _w((2, 64, 64)),
                  _w((2, 64, 64)), _w((2, 1, 64)), _w((2, 64, 64))],
        out_specs=[_b2(128)],
        out_shape=[jax.ShapeDtypeStruct((NP, 128), F32)],
    )(mt, h, rh, ma, m4, z, w["wh_x"], w["wh_h"], w["uh_x"], w["uh_h"],
      w["bh"], wint)[0]


def _tc_cand_m_dec(mt, h, rh, ma, m4, z, w, wint, wout, bout):
    return pl.pallas_call(
        _cand_m_dec_body,
        grid=(GRID,),
        in_specs=[_b2(128), _b2(128), _b2(128), _b3(128), _b3(128), _b2(128),
                  _w((2, 64, 64)), _w((2, 64, 64)), _w((2, 64, 64)),
                  _w((2, 64, 64)), _w((2, 1, 64)), _w((2, 64, 64)),
                  _w((2, 64, 2)), _w((2, 1, 2))],
        out_specs=[_b2(128), _b2(4), _b2(16)],
        out_shape=[jax.ShapeDtypeStruct((NP, 128), F32),
                   jax.ShapeDtypeStruct((NP, 4), F32),
                   jax.ShapeDtypeStruct((NP, 16), F32)],
    )(mt, h, rh, ma, m4, z, w["wh_x"], w["wh_h"], w["uh_x"], w["uh_h"],
      w["bh"], wint, wout, bout)


def _gate_w(p, din):
    return {
        "wzr_x": jnp.concatenate([p["Wz"][:din], p["Wr"][:din]], 1),
        "wzr_h": jnp.concatenate([p["Wz"][din:], p["Wr"][din:]], 1),
        "uzr_x": jnp.concatenate([p["Uz"][:din], p["Ur"][:din]], 1),
        "uzr_h": jnp.concatenate([p["Uz"][din:], p["Ur"][din:]], 1),
        "bzr": jnp.concatenate([p["bz"], p["br"]]).reshape(1, 128),
        "wh_x": p["Wh"][:din],
        "wh_h": p["Wh"][din:],
        "uh_x": p["Uh"][:din],
        "uh_h": p["Uh"][din:],
        "bh": p["bh"].reshape(1, 64),
    }


def _stack_w(pod, pdo, din):
    wod, wdo = _gate_w(pod, din), _gate_w(pdo, din)
    return jax.tree.map(lambda a, b: jnp.stack([a, b]), wod, wdo)


def kernel(x_seq, edge_index, params):
    src = edge_index[0].astype(jnp.int32)
    dst = edge_index[1].astype(jnp.int32)
    pad = E_PAD - E
    psrc = (jnp.arange(pad, dtype=jnp.int32) * 97) % N
    pdst = N + (jnp.arange(pad, dtype=jnp.int32) % TRASH)
    srcg = jnp.concatenate([src, psrc]).reshape(NCHUNKS, CHUNK)
    dstg = jnp.concatenate([dst, pdst]).reshape(NCHUNKS, CHUNK)
    srcg64 = srcg.reshape(NCHUNKS * 2, 64)
    dstg64 = dstg.reshape(NCHUNKS * 2, 64)
    zz128 = jnp.zeros((ZR, 128), F32)
    zz16 = jnp.zeros((ZR, 16), F32)

    sce = lambda t: _SCE(t, srcg64, dstg64, zz128)
    sc16 = lambda t: _SC16(t, srcg, dstg, zz16)

    p = params
    ap = p["prelu_a"].reshape(1, 64)
    we0 = _stack_w(p["od"]["enc0"], p["do"]["enc0"], 2)
    we1 = _stack_w(p["od"]["enc1"], p["do"]["enc1"], 64)
    wd0 = _stack_w(p["od"]["dec0"], p["do"]["dec0"], 2)
    wd1 = _stack_w(p["od"]["dec1"], p["do"]["dec1"], 64)
    ie0 = jnp.stack([p["int_enc0"]["Wod"], p["int_enc0"]["Wdo"]])
    ie1 = jnp.stack([p["int_enc1"]["Wod"], p["int_enc1"]["Wdo"]])
    id0 = jnp.stack([p["int_dec0"]["Wod"], p["int_dec0"]["Wdo"]])
    id1 = jnp.stack([p["int_dec1"]["Wod"], p["int_dec1"]["Wdo"]])
    wout = jnp.stack([p["od"]["Wout"], p["do"]["Wout"]])
    bout = jnp.stack([p["od"]["bout"].reshape(1, 2), p["do"]["bout"].reshape(1, 2)])

    xp = jnp.pad(x_seq, ((0, 0), (0, NP - N), (0, 0)))
    xs = jnp.moveaxis(xp, 0, 1).reshape(NP, 16)
    axp = sc16(xs)
    axm = axp[0] + axp[1]

    z2 = jnp.zeros((NP, 128), F32)
    zp = jnp.zeros((2, NP, 128), F32)
    h0, h1 = z2, z2
    for t in range(SEQ):
        xt = xp[t]
        x2 = jnp.concatenate([xt, xt], 1)
        axt = axm[:, 2 * t:2 * t + 2]
        ax2 = jnp.concatenate([axt, axt], 1)
        m1 = zp if t == 0 else sce(h0)
        rh, z = _tc_gates_x(x2, ax2, h0, m1, we0)
        m2 = zp if t == 0 else sce(rh)
        h0, mt = _tc_cand_x(x2, ax2, h0, rh, m2, z, we0, ie0, ap)
        m3a = sce(mt)
        m3b = zp if t == 0 else sce(h1)
        rh1, z1 = _tc_gates_m(mt, h1, m3a, m3b, we1)
        m4 = zp if t == 0 else sce(rh1)
        h1 = _tc_cand_m_enc(mt, h1, rh1, m3a, m4, z1, we1, ie1)

    dec = jnp.zeros((NP, 4), F32)
    s1t8 = None
    preds = []
    for t in range(HOR):
        m1 = sce(h0)
        if t == 0:
            ax2 = jnp.zeros((NP, 4), F32)
        else:
            axp2 = sc16(s1t8)
            axd = axp2[0] + axp2[1]
            ax2 = jnp.concatenate([axd[:, 0:2], axd[:, 8:10]], 1)
        rh, z = _tc_gates_x(dec, ax2, h0, m1, wd0)
        m2 = sce(rh)
        h0, mt = _tc_cand_x(dec, ax2, h0, rh, m2, z, wd0, id0, ap)
        m3a = sce(mt)
        m3b = sce(h1)
        rh1, z1 = _tc_gates_m(mt, h1, m3a, m3b, wd1)
        m4 = sce(rh1)
        h1, out, s1t8 = _tc_cand_m_dec(mt, h1, rh1, m3a, m4, z1, wd1, id1,
                                       wout, bout)
        dec = out
        preds.append(out)

    pod = jnp.stack([o[:N, 0:2] for o in preds])
    pdo = jnp.stack([o[:N, 2:4] for o in preds])
    return jnp.stack([pod, pdo])


# speedup vs baseline: 1.0034x; 1.0034x over previous
import jax
import jax.numpy as jnp
from jax import lax
from jax.experimental import pallas as pl
from jax.experimental.pallas import tpu as pltpu
from jax.experimental.pallas import tpu_sc as plsc

N = 10000
NP = 10240
E = 320000
SEQ = 8
HOR = 4
NC, NS = 2, 16
CHUNK = 128
CH = 160
NCHUNKS = NS * CH
E_PAD = NCHUNKS * CHUNK
CHE = NCHUNKS // (NC * NS)
RPT = NP // NS
ZR = NP // NS
TRASH = NP - N
BN = 2048
GRID = NP // BN

F32 = jnp.float32


def _make_sc_edge():
    mesh = plsc.VectorSubcoreMesh(core_axis_name="c", subcore_axis_name="s")
    CK = 64
    CPW = E_PAD // (NC * NS * CK)
    GRP = CPW // 8

    def body(table, srcg, dstg, zz, out, acc_sp, idx_v, rows_v, semg, semi):
        c = lax.axis_index("c")
        s = lax.axis_index("s")
        pltpu.sync_copy(zz, acc_sp.at[pl.ds(s * ZR, ZR), :])
        plsc.subcore_barrier()
        base = (c * NS + s) * CPW

        def gath(ring, row, buf):
            pltpu.async_copy(table.at[idx_v.at[ring, 0, row]], rows_v.at[buf], semg)

        def gath_wait(buf):
            pltpu.make_async_copy(table.at[idx_v.at[0, 0, 0]], rows_v.at[buf],
                                  semg).wait()

        pltpu.sync_copy(srcg.at[pl.ds(base, 8), :], idx_v.at[0, 0])
        pltpu.sync_copy(dstg.at[pl.ds(base, 8), :], idx_v.at[0, 1])
        gath(0, 0, 0)
        gath(0, 1, 1)
        gath(0, 2, 2)
        gath(0, 3, 3)

        @pl.loop(0, GRP)
        def _(g):
            r = lax.rem(g, 2)

            @pl.when(g < GRP - 1)
            def _():
                nb = base + (g + 1) * 8
                pltpu.async_copy(srcg.at[pl.ds(nb, 8), :], idx_v.at[1 - r, 0], semi)
                pltpu.async_copy(dstg.at[pl.ds(nb, 8), :], idx_v.at[1 - r, 1], semi)

            for j in range(8):
                b = j % 4
                gath_wait(b)
                pltpu.sync_copy(rows_v.at[b], acc_sp.at[idx_v.at[r, 1, j]], add=True)
                if j < 4:
                    gath(r, j + 4, b)
                else:
                    @pl.when(g < GRP - 1)
                    def _():
                        if j == 4:
                            pltpu.make_async_copy(srcg.at[pl.ds(base, 8), :],
                                                  idx_v.at[1 - r, 0], semi).wait()
                            pltpu.make_async_copy(dstg.at[pl.ds(base, 8), :],
                                                  idx_v.at[1 - r, 1], semi).wait()
                        gath(1 - r, j - 4, b)

        plsc.subcore_barrier()
        pltpu.sync_copy(acc_sp.at[pl.ds(s * RPT, RPT), :],
                        out.at[c, pl.ds(s * RPT, RPT), :])

    return pl.kernel(
        body,
        out_type=jax.ShapeDtypeStruct((NC, NP, 128), F32),
        mesh=mesh,
        scratch_types=[
            pltpu.VMEM_SHARED((NP, 128), F32),
            pltpu.VMEM((2, 2, 8, 64), jnp.int32),
            pltpu.VMEM((4, 64, 128), F32),
            pltpu.SemaphoreType.DMA,
            pltpu.SemaphoreType.DMA,
        ],
    )


def _make_sc_staged(wc):
    mesh = plsc.VectorSubcoreMesh(core_axis_name="c", subcore_axis_name="s")
    GRP = CH // 8

    def body(table, srcg, dstg, zz, out, tab_sp, acc_sp, idx_v, rows_v, semg, semi):
        c = lax.axis_index("c")
        s = lax.axis_index("s")
        pltpu.sync_copy(table.at[c, pl.ds(s * RPT, RPT), :],
                        tab_sp.at[pl.ds(s * RPT, RPT), :])
        pltpu.sync_copy(zz, acc_sp.at[pl.ds(s * ZR, ZR), :])
        plsc.subcore_barrier()
        base = s * CH
        pltpu.sync_copy(srcg.at[pl.ds(base, 8), :], idx_v.at[0, 0])
        pltpu.sync_copy(dstg.at[pl.ds(base, 8), :], idx_v.at[0, 1])

        @pl.loop(0, GRP)
        def _(g):
            r = lax.rem(g, 2)

            @pl.when(g > 0)
            def _():
                pltpu.make_async_copy(srcg.at[pl.ds(base, 8), :],
                                      idx_v.at[r, 0], semi).wait()
                pltpu.make_async_copy(dstg.at[pl.ds(base, 8), :],
                                      idx_v.at[r, 1], semi).wait()

            pltpu.async_copy(tab_sp.at[idx_v.at[r, 0, 0]], rows_v.at[0], semg)

            @pl.when(g < GRP - 1)
            def _():
                nb = base + (g + 1) * 8
                pltpu.async_copy(srcg.at[pl.ds(nb, 8), :], idx_v.at[1 - r, 0], semi)
                pltpu.async_copy(dstg.at[pl.ds(nb, 8), :], idx_v.at[1 - r, 1], semi)

            for j in range(8):
                b = j % 2
                pltpu.make_async_copy(tab_sp.at[idx_v.at[r, 0, j]],
                                      rows_v.at[b], semg).wait()
                if j < 7:
                    pltpu.async_copy(tab_sp.at[idx_v.at[r, 0, j + 1]],
                                     rows_v.at[1 - b], semg)
                pltpu.sync_copy(rows_v.at[b], acc_sp.at[idx_v.at[r, 1, j]], add=True)

        plsc.subcore_barrier()
        pltpu.sync_copy(acc_sp.at[pl.ds(s * RPT, RPT), :],
                        out.at[c, pl.ds(s * RPT, RPT), :])

    return pl.kernel(
        body,
        out_type=jax.ShapeDtypeStruct((NC, NP, wc), F32),
        mesh=mesh,
        scratch_types=[
            pltpu.VMEM_SHARED((NP, wc), F32),
            pltpu.VMEM_SHARED((NP, wc), F32),
            pltpu.VMEM((2, 2, 8, CHUNK), jnp.int32),
            pltpu.VMEM((2, CHUNK, wc), F32),
            pltpu.SemaphoreType.DMA,
            pltpu.SemaphoreType.DMA,
        ],
    )


_SCE = _make_sc_edge()
_SC8 = _make_sc_staged(8)


def _b3(k):
    return pl.BlockSpec((2, BN, k), lambda i: (0, i, 0))


def _b2(k):
    return pl.BlockSpec((BN, k), lambda i: (i, 0))


def _w(shape):
    nd = len(shape)
    return pl.BlockSpec(shape, lambda i, _n=nd: (0,) * _n)


def _dot(a, b):
    return jnp.dot(a, b, preferred_element_type=F32)


def _sl(v, b):
    return v[:, 64 * b:64 * b + 64]


def _gates_x_body(x, ax, h, m, wzx, wzh, uzx, uzh, bzr, rh, z):
    mm = m[0] + m[1]
    hv = h[...]
    for b in range(2):
        xb = x[:, 2 * b:2 * b + 2]
        axb = ax[:, 2 * b:2 * b + 2]
        zr = jax.nn.sigmoid(_dot(xb, wzx[b]) + _dot(_sl(hv, b), wzh[b])
                            + _dot(axb, uzx[b]) + _dot(_sl(mm, b), uzh[b]) + bzr[b])
        z[:, 64 * b:64 * b + 64] = zr[:, :64]
        rh[:, 64 * b:64 * b + 64] = zr[:, 64:] * _sl(hv, b)


def _gates_m_body(mt, h, ma, mb, wzm, wzh, uzm, uzh, bzr, rh, z):
    maa = ma[0] + ma[1]
    mbb = mb[0] + mb[1]
    mtv = mt[...]
    hv = h[...]
    for b in range(2):
        zr = jax.nn.sigmoid(_dot(_sl(mtv, b), wzm[b]) + _dot(_sl(hv, b), wzh[b])
                            + _dot(_sl(maa, b), uzm[b]) + _dot(_sl(mbb, b), uzh[b])
                            + bzr[b])
        z[:, 64 * b:64 * b + 64] = zr[:, :64]
        rh[:, 64 * b:64 * b + 64] = zr[:, 64:] * _sl(hv, b)


def _cand_x_body(x, ax, h, rh, m2, z, whx, whh, uhx, uhh, bh, wint, ap, hn_out, mt_out):
    m22 = m2[0] + m2[1]
    hv = h[...]
    rhv = rh[...]
    zv = z[...]
    hns = []
    for b in range(2):
        xb = x[:, 2 * b:2 * b + 2]
        axb = ax[:, 2 * b:2 * b + 2]
        hh = jnp.tanh(_dot(xb, whx[b]) + _dot(_sl(rhv, b), whh[b])
                      + _dot(axb, uhx[b]) + _dot(_sl(m22, b), uhh[b]) + bh[b])
        zb = _sl(zv, b)
        hns.append(zb * _sl(hv, b) + (1.0 - zb) * hh)
    for b in range(2):
        ib = jnp.tanh(_dot(hns[1 - b], wint[b]))
        v = hns[b] + ib
        hn_out[:, 64 * b:64 * b + 64] = v
        mt_out[:, 64 * b:64 * b + 64] = jnp.where(v >= 0, v, ap[...] * v)


def _cand_m_enc_body(mt, h, rh, ma, m4, z, whm, whh, uhm, uhh, bh, wint, hn_out):
    maa = ma[0] + ma[1]
    m44 = m4[0] + m4[1]
    mtv = mt[...]
    hv = h[...]
    rhv = rh[...]
    zv = z[...]
    hns = []
    for b in range(2):
        hh = jnp.tanh(_dot(_sl(mtv, b), whm[b]) + _dot(_sl(rhv, b), whh[b])
                      + _dot(_sl(maa, b), uhm[b]) + _dot(_sl(m44, b), uhh[b]) + bh[b])
        zb = _sl(zv, b)
        hns.append(zb * _sl(hv, b) + (1.0 - zb) * hh)
    for b in range(2):
        hn_out[:, 64 * b:64 * b + 64] = hns[b] + jnp.tanh(_dot(hns[1 - b], wint[b]))


def _cand_m_dec_body(mt, h, rh, ma, m4, z, whm, whh, uhm, uhh, bh, wint,
                     wout, bout, hn_out, out, s1t):
    maa = ma[0] + ma[1]
    m44 = m4[0] + m4[1]
    mtv = mt[...]
    hv = h[...]
    rhv = rh[...]
    zv = z[...]
    hns = []
    for b in range(2):
        hh = jnp.tanh(_dot(_sl(mtv, b), whm[b]) + _dot(_sl(rhv, b), whh[b])
                      + _dot(_sl(maa, b), uhm[b]) + _dot(_sl(m44, b), uhh[b]) + bh[b])
        zb = _sl(zv, b)
        hns.append(zb * _sl(hv, b) + (1.0 - zb) * hh)
    for b in range(2):
        v = hns[b] + jnp.tanh(_dot(hns[1 - b], wint[b]))
        hn_out[:, 64 * b:64 * b + 64] = v
        o = _dot(v, wout[b]) + bout[b]
        out[:, 2 * b:2 * b + 2] = o
        s1t[b, :, 0:2] = o
        s1t[b, :, 2:8] = jnp.zeros((s1t.shape[1], 6), F32)


def _tc_gates_x(x, ax, h, m, w):
    return pl.pallas_call(
        _gates_x_body,
        grid=(GRID,),
        in_specs=[_b2(4), _b2(4), _b2(128), _b3(128),
                  _w((2, 2, 128)), _w((2, 64, 128)), _w((2, 2, 128)),
                  _w((2, 64, 128)), _w((2, 1, 128))],
        out_specs=[_b2(128), _b2(128)],
        out_shape=[jax.ShapeDtypeStruct((NP, 128), F32),
                   jax.ShapeDtypeStruct((NP, 128), F32)],
    )(x, ax, h, m, w["wzr_x"], w["wzr_h"], w["uzr_x"], w["uzr_h"], w["bzr"])


def _tc_gates_m(mt, h, ma, mb, w):
    return pl.pallas_call(
        _gates_m_body,
        grid=(GRID,),
        in_specs=[_b2(128), _b2(128), _b3(128), _b3(128),
                  _w((2, 64, 128)), _w((2, 64, 128)), _w((2, 64, 128)),
                  _w((2, 64, 128)), _w((2, 1, 128))],
        out_specs=[_b2(128), _b2(128)],
        out_shape=[jax.ShapeDtypeStruct((NP, 128), F32),
                   jax.ShapeDtypeStruct((NP, 128), F32)],
    )(mt, h, ma, mb, w["wzr_x"], w["wzr_h"], w["uzr_x"], w["uzr_h"], w["bzr"])


def _tc_cand_x(x, ax, h, rh, m2, z, w, wint, ap):
    return pl.pallas_call(
        _cand_x_body,
        grid=(GRID,),
        in_specs=[_b2(4), _b2(4), _b2(128), _b2(128), _b3(128), _b2(128),
                  _w((2, 2, 64)), _w((2, 64, 64)), _w((2, 2, 64)),
                  _w((2, 64, 64)), _w((2, 1, 64)), _w((2, 64, 64)), _w((1, 64))],
        out_specs=[_b2(128), _b2(128)],
        out_shape=[jax.ShapeDtypeStruct((NP, 128), F32),
                   jax.ShapeDtypeStruct((NP, 128), F32)],
    )(x, ax, h, rh, m2, z, w["wh_x"], w["wh_h"], w["uh_x"], w["uh_h"],
      w["bh"], wint, ap)


def _tc_cand_m_enc(mt, h, rh, ma, m4, z, w, wint):
    return pl.pallas_call(
        _cand_m_enc_body,
        grid=(GRID,),
        in_specs=[_b2(128), _b2(128), _b2(128), _b3(128), _b3(128), _b2(128),
                  _w((2, 64, 64)), _w((2, 64, 64)), _w((2, 64, 64)),
                  _w((2, 64, 64)), _w((2, 1, 64)), _w((2, 64, 64))],
        out_specs=[_b2(128)],
        out_shape=[jax.ShapeDtypeStruct((NP, 128), F32)],
    )(mt, h, rh, ma, m4, z, w["wh_x"], w["wh_h"], w["uh_x"], w["uh_h"],
      w["bh"], wint)[0]


def _tc_cand_m_dec(mt, h, rh, ma, m4, z, w, wint, wout, bout):
    return pl.pallas_call(
        _cand_m_dec_body,
        grid=(GRID,),
        in_specs=[_b2(128), _b2(128), _b2(128), _b3(128), _b3(128), _b2(128),
                  _w((2, 64, 64)), _w((2, 64, 64)), _w((2, 64, 64)),
                  _w((2, 64, 64)), _w((2, 1, 64)), _w((2, 64, 64)),
                  _w((2, 64, 2)), _w((2, 1, 2))],
        out_specs=[_b2(128), _b2(4), _b3(8)],
        out_shape=[jax.ShapeDtypeStruct((NP, 128), F32),
                   jax.ShapeDtypeStruct((NP, 4), F32),
                   jax.ShapeDtypeStruct((2, NP, 8), F32)],
    )(mt, h, rh, ma, m4, z, w["wh_x"], w["wh_h"], w["uh_x"], w["uh_h"],
      w["bh"], wint, wout, bout)


def _gate_w(p, din):
    return {
        "wzr_x": jnp.concatenate([p["Wz"][:din], p["Wr"][:din]], 1),
        "wzr_h": jnp.concatenate([p["Wz"][din:], p["Wr"][din:]], 1),
        "uzr_x": jnp.concatenate([p["Uz"][:din], p["Ur"][:din]], 1),
        "uzr_h": jnp.concatenate([p["Uz"][din:], p["Ur"][din:]], 1),
        "bzr": jnp.concatenate([p["bz"], p["br"]]).reshape(1, 128),
        "wh_x": p["Wh"][:din],
        "wh_h": p["Wh"][din:],
        "uh_x": p["Uh"][:din],
        "uh_h": p["Uh"][din:],
        "bh": p["bh"].reshape(1, 64),
    }


def _stack_w(pod, pdo, din):
    wod, wdo = _gate_w(pod, din), _gate_w(pdo, din)
    return jax.tree.map(lambda a, b: jnp.stack([a, b]), wod, wdo)


def kernel(x_seq, edge_index, params):
    src = edge_index[0].astype(jnp.int32)
    dst = edge_index[1].astype(jnp.int32)
    pad = E_PAD - E
    psrc = (jnp.arange(pad, dtype=jnp.int32) * 97) % N
    pdst = N + (jnp.arange(pad, dtype=jnp.int32) % TRASH)
    srcg = jnp.concatenate([src, psrc]).reshape(NCHUNKS, CHUNK)
    dstg = jnp.concatenate([dst, pdst]).reshape(NCHUNKS, CHUNK)
    srcg64 = srcg.reshape(NCHUNKS * 2, 64)
    dstg64 = dstg.reshape(NCHUNKS * 2, 64)
    zz128 = jnp.zeros((ZR, 128), F32)
    zz8 = jnp.zeros((ZR, 8), F32)

    sce = lambda t: _SCE(t, srcg64, dstg64, zz128)
    sc8 = lambda t: _SC8(t, srcg, dstg, zz8)

    p = params
    ap = p["prelu_a"].reshape(1, 64)
    we0 = _stack_w(p["od"]["enc0"], p["do"]["enc0"], 2)
    we1 = _stack_w(p["od"]["enc1"], p["do"]["enc1"], 64)
    wd0 = _stack_w(p["od"]["dec0"], p["do"]["dec0"], 2)
    wd1 = _stack_w(p["od"]["dec1"], p["do"]["dec1"], 64)
    ie0 = jnp.stack([p["int_enc0"]["Wod"], p["int_enc0"]["Wdo"]])
    ie1 = jnp.stack([p["int_enc1"]["Wod"], p["int_enc1"]["Wdo"]])
    id0 = jnp.stack([p["int_dec0"]["Wod"], p["int_dec0"]["Wdo"]])
    id1 = jnp.stack([p["int_dec1"]["Wod"], p["int_dec1"]["Wdo"]])
    wout = jnp.stack([p["od"]["Wout"], p["do"]["Wout"]])
    bout = jnp.stack([p["od"]["bout"].reshape(1, 2), p["do"]["bout"].reshape(1, 2)])

    xp = jnp.pad(x_seq, ((0, 0), (0, NP - N), (0, 0)))
    xs = jnp.moveaxis(xp, 0, 1).reshape(NP, 16)
    axm = sc8(jnp.stack([xs[:, :8], xs[:, 8:]]))

    z2 = jnp.zeros((NP, 128), F32)
    zp = jnp.zeros((2, NP, 128), F32)
    h0, h1 = z2, z2
    for t in range(SEQ):
        xt = xp[t]
        x2 = jnp.concatenate([xt, xt], 1)
        axt = axm[t // 4, :, (t % 4) * 2:(t % 4) * 2 + 2]
        ax2 = jnp.concatenate([axt, axt], 1)
        m1 = zp if t == 0 else sce(h0)
        rh, z = _tc_gates_x(x2, ax2, h0, m1, we0)
        m2 = zp if t == 0 else sce(rh)
        h0, mt = _tc_cand_x(x2, ax2, h0, rh, m2, z, we0, ie0, ap)
        m3a = sce(mt)
        m3b = zp if t == 0 else sce(h1)
        rh1, z1 = _tc_gates_m(mt, h1, m3a, m3b, we1)
        m4 = zp if t == 0 else sce(rh1)
        h1 = _tc_cand_m_enc(mt, h1, rh1, m3a, m4, z1, we1, ie1)

    dec = jnp.zeros((NP, 4), F32)
    s1t8 = None
    preds = []
    for t in range(HOR):
        m1 = sce(h0)
        if t == 0:
            ax2 = jnp.zeros((NP, 4), F32)
        else:
            axd = sc8(s1t8)
            ax2 = jnp.concatenate([axd[0, :, 0:2], axd[1, :, 0:2]], 1)
        rh, z = _tc_gates_x(dec, ax2, h0, m1, wd0)
        m2 = sce(rh)
        h0, mt = _tc_cand_x(dec, ax2, h0, rh, m2, z, wd0, id0, ap)
        m3a = sce(mt)
        m3b = sce(h1)
        rh1, z1 = _tc_gates_m(mt, h1, m3a, m3b, wd1)
        m4 = sce(rh1)
        h1, out, s1t8 = _tc_cand_m_dec(mt, h1, rh1, m3a, m4, z1, wd1, id1,
                                       wout, bout)
        dec = out
        preds.append(out)

    pod = jnp.stack([o[:N, 0:2] for o in preds])
    pdo = jnp.stack([o[:N, 2:4] for o in preds])
    return jnp.stack([pod, pdo])
